# baseline (speedup 1.0000x reference)
import jax
import jax.numpy as jnp
from jax import lax
from jax.experimental import pallas as pl
from jax.experimental.pallas import tpu as pltpu
from jax.experimental.pallas import tpu_sc as plsc

N = 10000
E = 320000
D = 128
NC, NS, LANES = 2, 16, 16
NW = NC * NS
TILE = 128
NTILES = E // TILE
TPW = NTILES // NW
CHUNK = TPW * TILE
REM = NTILES - TPW * NW
STEPS = CHUNK // LANES
GROUP = 26
NPAD = 10240
NPAR = 64


def _sc_body(edge_hbm, opv_hbm, cntf_hbm, cntb_hbm,
             ed_v, ed2_v, cntf_v, cntb_v, opv_v, sem):
    c = lax.axis_index("c")
    s = lax.axis_index("s")
    wid = s * NC + c
    base = wid * CHUNK
    cps = [
        pltpu.async_copy(edge_hbm.at[:, pl.ds(base, CHUNK)], ed_v, sem),
        pltpu.async_copy(opv_hbm, opv_v, sem),
    ]
    z16 = jnp.zeros((LANES,), jnp.float32)

    def zstep(i, carry):
        cntf_v[pl.ds(i * LANES, LANES)] = z16
        cntb_v[pl.ds(i * LANES, LANES)] = z16
        return carry

    lax.fori_loop(0, NPAD // LANES, zstep, 0)
    for cp in cps:
        cp.wait()
    opvec = opv_v[...]
    ones = jnp.ones((LANES,), jnp.float32)
    lane_iota = lax.iota(jnp.int32, LANES)

    def scatter_vreg(s16, d16, mf, mb):
        @pl.when(jnp.sum((mf | mb).astype(jnp.int32)) > 0)
        def _():
            def jstep(j, carry):
                lane = lane_iota == j
                plsc.addupdate_scatter(cntf_v, [s16], ones, mask=mf & lane)
                plsc.addupdate_scatter(cntb_v, [d16], ones, mask=mb & lane)
                return carry

            lax.fori_loop(0, LANES, jstep, 0)

    def scan_vreg(ref, off):
        s16 = ref[0, pl.ds(off, LANES)]
        d16 = ref[1, pl.ds(off, LANES)]
        scatter_vreg(s16, d16, d16 == opvec, s16 == opvec)

    def group_step(g, carry):
        gbase = g * (GROUP * LANES)
        hit = jnp.zeros((LANES,), jnp.bool_)
        for k in range(GROUP):
            s16 = ed_v[0, pl.ds(gbase + k * LANES, LANES)]
            d16 = ed_v[1, pl.ds(gbase + k * LANES, LANES)]
            hit = hit | (s16 == opvec) | (d16 == opvec)

        @pl.when(jnp.sum(hit.astype(jnp.int32)) > 0)
        def _():
            def kstep(k, carry2):
                scan_vreg(ed_v, gbase + k * LANES)
                return carry2

            lax.fori_loop(0, GROUP, kstep, 0)

        return carry

    lax.fori_loop(0, STEPS // GROUP, group_step, 0)

    @pl.when(wid < REM)
    def _():
        pltpu.sync_copy(edge_hbm.at[:, pl.ds(NW * CHUNK + wid * TILE, TILE)],
                        ed2_v)

        def lstep(k, carry):
            scan_vreg(ed2_v, k * LANES)
            return carry

        lax.fori_loop(0, TILE // LANES, lstep, 0)

    wcs = [
        pltpu.async_copy(cntf_v, cntf_hbm.at[pl.ds(wid * NPAD, NPAD)], sem),
        pltpu.async_copy(cntb_v, cntb_hbm.at[pl.ds(wid * NPAD, NPAD)], sem),
    ]
    for cp in wcs:
        cp.wait()


def _make_sc_counts():
    return pl.kernel(
        _sc_body,
        out_type=(
            jax.ShapeDtypeStruct((NW * NPAD,), jnp.float32),
            jax.ShapeDtypeStruct((NW * NPAD,), jnp.float32),
        ),
        mesh=plsc.VectorSubcoreMesh(core_axis_name="c", subcore_axis_name="s",
                                    num_cores=NC, num_subcores=NS),
        scratch_types=[
            pltpu.VMEM((2, CHUNK), jnp.int32),
            pltpu.VMEM((2, TILE), jnp.int32),
            pltpu.VMEM((NPAD,), jnp.float32),
            pltpu.VMEM((NPAD,), jnp.float32),
            pltpu.VMEM((LANES,), jnp.int32),
            pltpu.SemaphoreType.DMA,
        ],
        compiler_params=pltpu.CompilerParams(needs_layout_passes=False,
                                             skip_device_barrier=True,
                                             disable_bounds_checks=True,
                                             disable_semaphore_checks=True),
        name="devnet_edge_counts_sc",
    )


def _dot_t(a, b):
    return lax.dot_general(a, b, (((1,), (1,)), ((), ())),
                           preferred_element_type=jnp.float32)


def _dot(a, b):
    return lax.dot_general(a, b, (((1,), (0,)), ((), ())),
                           preferred_element_type=jnp.float32)


def _tc_a_body(op_ref, par_ref, feat_ref, wf_ref, alf_ref, arf_ref,
               wb_ref, alb_ref, arb_ref, el_ref, aux_ref):
    feat = feat_ref[...]
    wl = jnp.concatenate([_dot_t(alf_ref[...], wf_ref[...]),
                          _dot_t(alb_ref[...], wb_ref[...])], axis=0)
    fop = feat_ref[pl.ds(op_ref[0], 1), :]
    er_f = _dot_t(fop, _dot_t(arf_ref[...], wf_ref[...]))
    er_b = _dot_t(fop, _dot_t(arb_ref[...], wb_ref[...]))
    el_ref[...] = _dot_t(wl, feat)

    def pstep(i, acc):
        return acc + feat_ref[pl.ds(par_ref[i], 1), :]

    para = lax.fori_loop(0, NPAR, pstep, jnp.zeros((1, D), jnp.float32))
    aux_ref[0:1, :] = fop
    aux_ref[1:2, :] = para
    aux_ref[2:3, :] = jnp.concatenate(
        [er_f, er_b, jnp.zeros((1, D - 2), jnp.float32)], axis=1)


def _tc_b_body(el_ref, aux_ref, feat_ref, cntf_ref, cntb_ref,
               wf_ref, bf_ref, wb_ref, bb_ref, out_ref):
    feat = feat_ref[...]
    er2 = jnp.concatenate([aux_ref[2:3, 0:1], aux_ref[2:3, 1:2]],
                          axis=0)
    x = el_ref[...] + er2
    v = jnp.where(x >= 0.0, x, 0.2 * x)
    cfp = cntf_ref[...]
    cbp = cntb_ref[...]
    cf1 = cfp[0:NPAD]
    cb1 = cbp[0:NPAD]
    for w in range(1, NW):
        cf1 = cf1 + cfp[w * NPAD:(w + 1) * NPAD]
        cb1 = cb1 + cbp[w * NPAD:(w + 1) * NPAD]
    cf = cf1[:N].reshape(1, N)
    cb = cb1[:N].reshape(1, N)
    cnt = jnp.concatenate([cf, cb], axis=0)
    has = cnt > 0.0
    vm = jnp.where(has, v, -jnp.inf)
    m = jnp.max(vm, axis=1, keepdims=True)
    m0 = jnp.where(jnp.isfinite(m), m, 0.0)
    numer = jnp.where(has, cnt * jnp.exp(vm - m0), 0.0)
    den = jnp.sum(numer, axis=1, keepdims=True)
    wgt = numer / jnp.maximum(den, 1e-16)
    pre = _dot(wgt, feat)
    out_ref[0:1, :] = _dot(pre[0:1, :], wf_ref[...]) + bf_ref[...]
    out_ref[1:2, :] = _dot(pre[1:2, :], wb_ref[...]) + bb_ref[...]
    out_ref[2:3, :] = aux_ref[0:1, :]
    out_ref[3:4, :] = aux_ref[1:2, :]


def kernel(feat, edge_index, op, parallel, W_f, attn_l_f, attn_r_f, bias_f,
           W_b, attn_l_b, attn_r_b, bias_b):
    op32 = jnp.asarray(op, jnp.int32)
    opv = jnp.full((LANES,), op32, jnp.int32)
    cntf, cntb = _make_sc_counts()(edge_index.astype(jnp.int32), opv)
    el, aux = pl.pallas_call(
        _tc_a_body,
        out_shape=(jax.ShapeDtypeStruct((2, N), jnp.float32),
                   jax.ShapeDtypeStruct((3, D), jnp.float32)),
        in_specs=[pl.BlockSpec(memory_space=pltpu.SMEM),
                  pl.BlockSpec(memory_space=pltpu.SMEM)] +
                 [pl.BlockSpec()] * 7,
        compiler_params=pltpu.CompilerParams(skip_device_barrier=True),
        name="devnet_dense_tc_a",
    )(op32.reshape(1), parallel.astype(jnp.int32), feat,
      W_f, attn_l_f.reshape(1, D), attn_r_f.reshape(1, D),
      W_b, attn_l_b.reshape(1, D), attn_r_b.reshape(1, D))
    out4 = pl.pallas_call(
        _tc_b_body,
        out_shape=jax.ShapeDtypeStruct((4, D), jnp.float32),
        compiler_params=pltpu.CompilerParams(skip_device_barrier=True),
        name="devnet_dense_tc_b",
    )(el, aux, feat, cntf, cntb,
      W_f, bias_f.reshape(1, D), W_b, bias_b.reshape(1, D))
    return out4.reshape(4 * D)
